# baseline (speedup 1.0000x reference)
import functools

import jax
import jax.numpy as jnp
from jax import lax
from jax.experimental import pallas as pl
from jax.experimental.pallas import tpu as pltpu
from jax.experimental.pallas import tpu_sc as plsc

NUM_EMB = 1000000
DIM = 32
ROWS128 = NUM_EMB * DIM // 128
BLK = 2000
N_BLOCKS = ROWS128 // BLK

BATCH = 16384
HIST = 50
NW = 32
B_PER_W = BATCH // NW
NB = 16
N_CHUNKS = B_PER_W // NB


def _maxabs_body(x_ref, o_ref):
    i = pl.program_id(0)

    @pl.when(i == 0)
    def _init():
        o_ref[...] = jnp.zeros((1, 1), jnp.float32)

    o_ref[...] = jnp.maximum(o_ref[...], jnp.max(jnp.abs(x_ref[...])))


def _quant_body(s_ref, x_ref, o_ref):
    scale = jnp.maximum(s_ref[...] / 7.0, 1e-08)
    o_ref[...] = jnp.clip(jnp.round(x_ref[...] / scale), -8.0, 7.0) * scale


@functools.cache
def _make_gather():
    mesh = plsc.VectorSubcoreMesh(core_axis_name="c", subcore_axis_name="s")

    @functools.partial(
        pl.kernel,
        mesh=mesh,
        compiler_params=pltpu.CompilerParams(use_tc_tiling_on_sc=False, needs_layout_passes=False),
        out_type=jax.ShapeDtypeStruct((HIST, DIM, BATCH), jnp.float32),
        scratch_types=[
            pltpu.VMEM((NB, HIST), jnp.int32),
            pltpu.VMEM((NB, HIST, DIM), jnp.float32),
            pltpu.VMEM((HIST, DIM, NB), jnp.float32),
            pltpu.SemaphoreType.DMA,
        ],
    )
    def gather_k(table_hbm, idx_hbm, out_hbm, idx_v, rows_v, t_v, sem):
        wid = lax.axis_index("s") * 2 + lax.axis_index("c")
        base = wid * B_PER_W
        lane = lax.iota(jnp.int32, 16)
        zero16 = jnp.zeros((16,), jnp.int32)

        def chunk(g, _):
            i0 = base + g * NB
            pltpu.sync_copy(idx_hbm.at[pl.ds(i0, NB)], idx_v)
            for b in range(NB):
                pltpu.async_copy(
                    table_hbm.at[idx_v.at[b]], rows_v.at[b], sem
                )
            for b in range(NB):
                pltpu.make_async_copy(
                    table_hbm.at[idx_v.at[b]], rows_v.at[b], sem
                ).wait()

            def trow(j, _c):
                for k in range(DIM):
                    v = plsc.load_gather(
                        rows_v, [lane, zero16 + j, zero16 + k]
                    )
                    t_v[j, k, pl.ds(0, 16)] = v
                return 0

            lax.fori_loop(0, HIST, trow, 0)
            pltpu.sync_copy(t_v, out_hbm.at[:, :, pl.ds(i0, NB)])
            return 0

        lax.fori_loop(0, N_CHUNKS, chunk, 0)

    return gather_k


def kernel(x, weight_fp):
    w128 = weight_fp.reshape(ROWS128, 128)

    maxabs = pl.pallas_call(
        _maxabs_body,
        grid=(N_BLOCKS,),
        in_specs=[pl.BlockSpec((BLK, 128), lambda i: (i, 0))],
        out_specs=pl.BlockSpec((1, 1), lambda i: (0, 0)),
        out_shape=jax.ShapeDtypeStruct((1, 1), jnp.float32),
    )(w128)

    w_q = pl.pallas_call(
        _quant_body,
        grid=(N_BLOCKS,),
        in_specs=[
            pl.BlockSpec((1, 1), lambda i: (0, 0)),
            pl.BlockSpec((BLK, 128), lambda i: (i, 0)),
        ],
        out_specs=pl.BlockSpec((BLK, 128), lambda i: (i, 0)),
        out_shape=jax.ShapeDtypeStruct((ROWS128, 128), jnp.float32),
    )(maxabs, w128)

    w_q = w_q.reshape(NUM_EMB, DIM)
    out = _make_gather()(w_q, x.astype(jnp.int32))
    return out.transpose(2, 0, 1)


# speedup vs baseline: 1.0313x; 1.0313x over previous
import functools

import jax
import jax.numpy as jnp
from jax import lax
from jax.experimental import pallas as pl
from jax.experimental.pallas import tpu as pltpu
from jax.experimental.pallas import tpu_sc as plsc

NUM_EMB = 1000000
DIM = 32
ROWS128 = NUM_EMB * DIM // 128
BLK = 2000
N_BLOCKS = ROWS128 // BLK

BATCH = 16384
HIST = 50
NW = 32
B_PER_W = BATCH // NW
NB = 32
N_CHUNKS = B_PER_W // NB
NJK = HIST * DIM
SCAT_G = 20
SCAT_R = NJK // SCAT_G


def _maxabs_body(x_ref, o_ref):
    i = pl.program_id(0)

    @pl.when(i == 0)
    def _init():
        o_ref[...] = jnp.zeros((1, 1), jnp.float32)

    o_ref[...] = jnp.maximum(o_ref[...], jnp.max(jnp.abs(x_ref[...])))


def _quant_body(s_ref, x_ref, o_ref):
    scale = jnp.maximum(s_ref[...] / 7.0, 1e-08)
    o_ref[...] = jnp.clip(jnp.round(x_ref[...] / scale), -8.0, 7.0) * scale


@functools.cache
def _make_gather():
    mesh = plsc.VectorSubcoreMesh(core_axis_name="c", subcore_axis_name="s")

    @functools.partial(
        pl.kernel,
        mesh=mesh,
        compiler_params=pltpu.CompilerParams(use_tc_tiling_on_sc=False, needs_layout_passes=False),
        out_type=jax.ShapeDtypeStruct((NJK * BATCH // DIM, DIM), jnp.float32),
        scratch_types=[
            pltpu.VMEM((NB, HIST), jnp.int32),
            pltpu.VMEM((NB, HIST, DIM), jnp.float32),
            pltpu.VMEM((NJK, DIM), jnp.float32),
            pltpu.VMEM((SCAT_G, SCAT_R), jnp.int32),
            pltpu.SemaphoreType.DMA,
            pltpu.SemaphoreType.DMA,
        ],
    )
    def gather_k(table_hbm, idx_hbm, out_hbm, idx_v, rows_v, t_v, si_v, sem, sem2):
        wid = lax.axis_index("s") * 2 + lax.axis_index("c")
        base = wid * B_PER_W
        lane = lax.iota(jnp.int32, 16)
        zero16 = jnp.zeros((16,), jnp.int32)

        def chunk(g, _):
            i0 = base + g * NB
            for a in range(SCAT_G):
                for mm in range(SCAT_R // 16):
                    r = a * SCAT_R + mm * 16
                    si_v[a, pl.ds(mm * 16, 16)] = (
                        (lane + r) * (BATCH // NB) + i0 // NB
                    )

            pltpu.sync_copy(idx_hbm.at[pl.ds(i0, NB)], idx_v)
            for b in range(NB):
                pltpu.async_copy(
                    table_hbm.at[idx_v.at[b]], rows_v.at[b], sem
                )
            for b in range(NB):
                pltpu.make_async_copy(
                    table_hbm.at[idx_v.at[b]], rows_v.at[b], sem
                ).wait()

            def trow(j, _c):
                r0 = j * DIM
                for k in range(DIM):
                    for h in range(NB // 16):
                        v = plsc.load_gather(
                            rows_v,
                            [lane + 16 * h, zero16 + j, zero16 + k],
                        )
                        t_v[r0 + k, pl.ds(16 * h, 16)] = v
                return 0

            lax.fori_loop(0, HIST, trow, 0)

            for a in range(SCAT_G):
                pltpu.async_copy(
                    t_v.at[pl.ds(a * SCAT_R, SCAT_R)],
                    out_hbm.at[si_v.at[a]],
                    sem2,
                )
            for a in range(SCAT_G):
                pltpu.make_async_copy(
                    t_v.at[pl.ds(a * SCAT_R, SCAT_R)],
                    out_hbm.at[si_v.at[a]],
                    sem2,
                ).wait()
            return 0

        lax.fori_loop(0, N_CHUNKS, chunk, 0)

    return gather_k


def kernel(x, weight_fp):
    w128 = weight_fp.reshape(ROWS128, 128)

    maxabs = pl.pallas_call(
        _maxabs_body,
        grid=(N_BLOCKS,),
        in_specs=[pl.BlockSpec((BLK, 128), lambda i: (i, 0))],
        out_specs=pl.BlockSpec((1, 1), lambda i: (0, 0)),
        out_shape=jax.ShapeDtypeStruct((1, 1), jnp.float32),
    )(w128)

    w_q = pl.pallas_call(
        _quant_body,
        grid=(N_BLOCKS,),
        in_specs=[
            pl.BlockSpec((1, 1), lambda i: (0, 0)),
            pl.BlockSpec((BLK, 128), lambda i: (i, 0)),
        ],
        out_specs=pl.BlockSpec((BLK, 128), lambda i: (i, 0)),
        out_shape=jax.ShapeDtypeStruct((ROWS128, 128), jnp.float32),
    )(maxabs, w128)

    w_q = w_q.reshape(NUM_EMB, DIM)
    out = _make_gather()(w_q, x.astype(jnp.int32))
    return out.reshape(HIST, DIM, BATCH).transpose(2, 0, 1)
